# Initial kernel scaffold; baseline (speedup 1.0000x reference)
import jax
import jax.numpy as jnp
from jax.experimental import pallas as pl


def kernel(xin, codebooks):
    raise NotImplementedError("write your pallas kernel here")


# speedup vs baseline: 1.3651x; 1.3651x over previous
import jax
import jax.numpy as jnp
from jax.experimental import pallas as pl

_N_CODES = 1024
_N_GROUPS = 2
_CODE_W = 512
_GROUP_DIM = _CODE_W // _N_GROUPS
_R_LAYERS = 8
_TB = 512


def _vq_kernel(x_ref, cb_ref, q_ref, idx_ref, loss_ref):
    res = x_ref[0]
    qacc = jnp.zeros_like(res)
    losses = []
    for l in range(_R_LAYERS):
        qparts = []
        for g in range(_N_GROUPS):
            xg = res[g * _GROUP_DIM:(g + 1) * _GROUP_DIM, :]
            w = cb_ref[l, g]
            sx = jnp.sum(xg * xg, axis=0, keepdims=True)
            sw = jnp.sum(w * w, axis=1, keepdims=True)
            mm = jax.lax.dot_general(
                w, xg, (((1,), (0,)), ((), ())),
                preferred_element_type=jnp.float32)
            d = (sx + sw) - 2.0 * mm
            minv = jnp.min(d, axis=0, keepdims=True)
            iota = jax.lax.broadcasted_iota(jnp.int32, d.shape, 0)
            idx = jnp.min(jnp.where(d == minv, iota, _N_CODES),
                          axis=0, keepdims=True)
            idx_ref[2 * l + g, :] = idx[0]
            oh = (iota == idx).astype(jnp.float32)
            qg = jax.lax.dot_general(
                w, oh, (((0,), (0,)), ((), ())),
                preferred_element_type=jnp.float32,
                precision=jax.lax.Precision.HIGHEST)
            qparts.append(qg)
        q = jnp.concatenate(qparts, axis=0)
        res = res - q
        qacc = qacc + q
        losses.append(jnp.sum(res * res))
    q_ref[0] = qacc
    loss_ref[0, 0, :] = jnp.stack(losses)


def kernel(xin, codebooks):
    b, c, t = xin.shape
    gt = t // _TB
    nblocks = b * gt
    q, idx, lossp = pl.pallas_call(
        _vq_kernel,
        grid=(b, gt),
        in_specs=[
            pl.BlockSpec((1, c, _TB), lambda i, j: (i, 0, j)),
            pl.BlockSpec(codebooks.shape, lambda i, j: (0, 0, 0, 0)),
        ],
        out_specs=[
            pl.BlockSpec((1, c, _TB), lambda i, j: (i, 0, j)),
            pl.BlockSpec((_N_GROUPS * _R_LAYERS, _TB),
                         lambda i, j: (0, i * (t // _TB) + j)),
            pl.BlockSpec((1, 1, _R_LAYERS),
                         lambda i, j: (i * (t // _TB) + j, 0, 0)),
        ],
        out_shape=[
            jax.ShapeDtypeStruct((b, c, t), jnp.float32),
            jax.ShapeDtypeStruct((_N_GROUPS * _R_LAYERS, b * t), jnp.int32),
            jax.ShapeDtypeStruct((nblocks, 1, _R_LAYERS), jnp.float32),
        ],
    )(xin, codebooks)
    ntot = b * c * t
    loss = jnp.mean(jnp.sum(lossp.reshape(nblocks, _R_LAYERS), axis=0)) * 1.25 / ntot
    return q, loss, idx


# speedup vs baseline: 2.9628x; 2.1703x over previous
import jax
import jax.numpy as jnp
from jax.experimental import pallas as pl

_N_CODES = 1024
_N_GROUPS = 2
_CODE_W = 512
_GROUP_DIM = _CODE_W // _N_GROUPS
_R_LAYERS = 8
_TB = 512


def _vq_kernel(x_ref, cb_ref, q_ref, idx_ref, loss_ref):
    res = x_ref[0]
    qacc = jnp.zeros_like(res)
    losses = []
    for l in range(_R_LAYERS):
        qparts = []
        for g in range(_N_GROUPS):
            xg = res[g * _GROUP_DIM:(g + 1) * _GROUP_DIM, :]
            w = cb_ref[l, g]
            sx = jnp.sum(xg * xg, axis=0, keepdims=True)
            sw = jnp.sum(w * w, axis=1, keepdims=True)
            mm = jax.lax.dot_general(
                w, xg, (((1,), (0,)), ((), ())),
                preferred_element_type=jnp.float32)
            d = (sx + sw) - 2.0 * mm
            minv = jnp.min(d, axis=0, keepdims=True)
            iota = jax.lax.broadcasted_iota(jnp.int32, d.shape, 0)
            idx = jnp.min(jnp.where(d == minv, iota, _N_CODES),
                          axis=0, keepdims=True)
            idx_ref[2 * l + g, :] = idx[0]
            oh = (iota == idx).astype(jnp.bfloat16)
            w_hi = w.astype(jnp.bfloat16)
            w_lo = (w - w_hi.astype(jnp.float32)).astype(jnp.bfloat16)
            dn = (((0,), (0,)), ((), ()))
            qg = (jax.lax.dot_general(w_hi, oh, dn,
                                      preferred_element_type=jnp.float32)
                  + jax.lax.dot_general(w_lo, oh, dn,
                                        preferred_element_type=jnp.float32))
            qparts.append(qg)
        q = jnp.concatenate(qparts, axis=0)
        res = res - q
        qacc = qacc + q
        losses.append(jnp.sum(res * res))
    q_ref[0] = qacc
    loss_ref[0, 0, :] = jnp.stack(losses)


def kernel(xin, codebooks):
    b, c, t = xin.shape
    gt = t // _TB
    nblocks = b * gt
    q, idx, lossp = pl.pallas_call(
        _vq_kernel,
        grid=(b, gt),
        in_specs=[
            pl.BlockSpec((1, c, _TB), lambda i, j: (i, 0, j)),
            pl.BlockSpec(codebooks.shape, lambda i, j: (0, 0, 0, 0)),
        ],
        out_specs=[
            pl.BlockSpec((1, c, _TB), lambda i, j: (i, 0, j)),
            pl.BlockSpec((_N_GROUPS * _R_LAYERS, _TB),
                         lambda i, j: (0, i * (t // _TB) + j)),
            pl.BlockSpec((1, 1, _R_LAYERS),
                         lambda i, j: (i * (t // _TB) + j, 0, 0)),
        ],
        out_shape=[
            jax.ShapeDtypeStruct((b, c, t), jnp.float32),
            jax.ShapeDtypeStruct((_N_GROUPS * _R_LAYERS, b * t), jnp.int32),
            jax.ShapeDtypeStruct((nblocks, 1, _R_LAYERS), jnp.float32),
        ],
    )(xin, codebooks)
    ntot = b * c * t
    loss = jnp.mean(jnp.sum(lossp.reshape(nblocks, _R_LAYERS), axis=0)) * 1.25 / ntot
    return q, loss, idx


# speedup vs baseline: 3.6979x; 1.2481x over previous
import jax
import jax.numpy as jnp
from jax.experimental import pallas as pl

_N_CODES = 1024
_N_GROUPS = 2
_CODE_W = 512
_GROUP_DIM = _CODE_W // _N_GROUPS
_R_LAYERS = 8
_TB = 512


def _vq_kernel(x_ref, w2_ref, sw_ref, q_ref, idx_ref, loss_ref):
    res = x_ref[0]
    qacc = jnp.zeros_like(res)
    losses = []
    for l in range(_R_LAYERS):
        qparts = []
        for g in range(_N_GROUPS):
            xg = res[g * _GROUP_DIM:(g + 1) * _GROUP_DIM, :]
            sx = jnp.sum(xg * xg, axis=0, keepdims=True)
            sw = sw_ref[l, g]
            mmneg = jax.lax.dot_general(
                w2_ref[l, g], xg, (((1,), (0,)), ((), ())),
                preferred_element_type=jnp.float32)
            d = (sx + sw) + mmneg
            minv = jnp.min(d, axis=0, keepdims=True)
            iota = jax.lax.broadcasted_iota(jnp.int32, d.shape, 0)
            idx = jnp.min(jnp.where(d == minv, iota, _N_CODES),
                          axis=0, keepdims=True)
            idx_ref[2 * l + g, :] = idx[0]
            oh = (iota == idx).astype(jnp.float32)
            qg2 = jax.lax.dot_general(
                w2_ref[l, g], oh, (((0,), (0,)), ((), ())),
                preferred_element_type=jnp.float32)
            qparts.append(qg2)
        q = jnp.concatenate(qparts, axis=0) * -0.5
        res = res - q
        qacc = qacc + q
        losses.append(jnp.sum(res * res))
    q_ref[0] = qacc
    loss_ref[0, 0, :] = jnp.stack(losses)


def kernel(xin, codebooks):
    b, c, t = xin.shape
    gt = t // _TB
    nblocks = b * gt
    w2 = -2.0 * codebooks
    sw = jnp.sum(codebooks ** 2, axis=3)[..., None]
    cb_spec = lambda shape: pl.BlockSpec(shape, lambda i, j: (0, 0, 0, 0))
    q, idx, lossp = pl.pallas_call(
        _vq_kernel,
        grid=(b, gt),
        in_specs=[
            pl.BlockSpec((1, c, _TB), lambda i, j: (i, 0, j)),
            cb_spec(w2.shape),
            cb_spec(sw.shape),
        ],
        out_specs=[
            pl.BlockSpec((1, c, _TB), lambda i, j: (i, 0, j)),
            pl.BlockSpec((_N_GROUPS * _R_LAYERS, _TB),
                         lambda i, j: (0, i * (t // _TB) + j)),
            pl.BlockSpec((1, 1, _R_LAYERS),
                         lambda i, j: (i * (t // _TB) + j, 0, 0)),
        ],
        out_shape=[
            jax.ShapeDtypeStruct((b, c, t), jnp.float32),
            jax.ShapeDtypeStruct((_N_GROUPS * _R_LAYERS, b * t), jnp.int32),
            jax.ShapeDtypeStruct((nblocks, 1, _R_LAYERS), jnp.float32),
        ],
    )(xin, w2, sw)
    ntot = b * c * t
    loss = jnp.mean(jnp.sum(lossp.reshape(nblocks, _R_LAYERS), axis=0)) * 1.25 / ntot
    return q, loss, idx
